# baseline (speedup 1.0000x reference)
import functools

import jax
import jax.numpy as jnp
from jax import lax
from jax.experimental import pallas as pl
from jax.experimental.pallas import tpu as pltpu
from jax.experimental.pallas import tpu_sc as plsc

NUM_CORES = 2
NUM_SUBCORES = 16
NUM_WORKERS = NUM_CORES * NUM_SUBCORES
LANES = 16
B, D, H, W = 2, 160, 192, 224
SPLIT = 8
HH = H // SPLIT
NUM_TILES = B * D * SPLIT
TILES_PER_WORKER = NUM_TILES // NUM_WORKERS
VECS_PER_ROW = W // LANES
NBUF = 8
TBL = 32


@functools.partial(
    pl.kernel,
    out_type=jax.ShapeDtypeStruct((B, 1, D, H, W), jnp.float32),
    mesh=plsc.VectorSubcoreMesh(
        core_axis_name="c",
        subcore_axis_name="s",
        num_cores=NUM_CORES,
        num_subcores=NUM_SUBCORES,
    ),
    scratch_types=[
        pltpu.VMEM((TBL, LANES), jnp.float32),
        pltpu.VMEM((TBL,), jnp.int32),
        pltpu.VMEM((TBL,), jnp.int32),
        [pltpu.VMEM((HH, W), jnp.float32) for _ in range(NBUF)],
        [pltpu.VMEM((HH, W), jnp.float32) for _ in range(NBUF)],
        [pltpu.SemaphoreType.DMA for _ in range(NBUF)],
        [pltpu.SemaphoreType.DMA for _ in range(NBUF)],
    ],
    compiler_params=pltpu.CompilerParams(needs_layout_passes=False),
)
def _remap(x_hbm, src_hbm, dst_hbm, out_hbm, tab_v, src_v, dst_v, inb, outb,
           in_sem, out_sem):
    wid = lax.axis_index("s") * NUM_CORES + lax.axis_index("c")
    tile0 = wid * TILES_PER_WORKER

    pltpu.sync_copy(src_hbm, src_v)
    pltpu.sync_copy(dst_hbm, dst_v)
    lane = lax.iota(jnp.int32, LANES)
    zero = jnp.zeros((LANES,), jnp.float32)
    for v in range(TBL):
        tab_v[v, pl.ds(0, LANES)] = zero
    for h in range(TBL // LANES):
        s = src_v[pl.ds(h * LANES, LANES)]
        d = dst_v[pl.ds(h * LANES, LANES)].astype(jnp.float32)
        for l in range(LANES):
            plsc.store_scatter(tab_v, [s, jnp.full((LANES,), l, jnp.int32)],
                               d)

    def tile_slice(ref, t):
        n = t // (D * SPLIT)
        r = t % (D * SPLIT)
        d = r // SPLIT
        h = (r % SPLIT) * HH
        return ref.at[n, 0, d, pl.ds(h, HH)]

    for b in range(NBUF):
        pltpu.async_copy(tile_slice(x_hbm, tile0 + b), inb[b], in_sem[b])

    @pl.loop(0, TILES_PER_WORKER, step=NBUF)
    def _(c0):
        for b in range(NBUF):
            c = c0 + b
            t = tile0 + c
            pltpu.make_async_copy(tile_slice(x_hbm, t), inb[b],
                                  in_sem[b]).wait()

            @pl.when(c >= NBUF)
            def _():
                pltpu.make_async_copy(outb[b], tile_slice(out_hbm, t),
                                      out_sem[b]).wait()

            src_buf = inb[b]
            dst_buf = outb[b]

            @plsc.parallel_loop(0, HH, 1, unroll=2)
            def _(r):
                for j in range(VECS_PER_ROW):
                    v = src_buf[r, pl.ds(j * LANES, LANES)]
                    idx = v.astype(jnp.int32)
                    dst_buf[r, pl.ds(j * LANES, LANES)] = plsc.load_gather(
                        tab_v, [idx, lane])

            pltpu.async_copy(dst_buf, tile_slice(out_hbm, t), out_sem[b])

            @pl.when(c + NBUF < TILES_PER_WORKER)
            def _():
                pltpu.async_copy(tile_slice(x_hbm, t + NBUF), inb[b],
                                 in_sem[b])

    for b in range(NBUF):
        t = tile0 + TILES_PER_WORKER - NBUF + b
        pltpu.make_async_copy(outb[b], tile_slice(out_hbm, t),
                              out_sem[b]).wait()


def kernel(x, source_values, dest_values):
    return _remap(x, source_values, dest_values)


# speedup vs baseline: 1.1938x; 1.1938x over previous
import functools

import jax
import jax.numpy as jnp
from jax import lax
from jax.experimental import pallas as pl
from jax.experimental.pallas import tpu as pltpu
from jax.experimental.pallas import tpu_sc as plsc

NUM_CORES = 2
NUM_SUBCORES = 16
NUM_WORKERS = NUM_CORES * NUM_SUBCORES
LANES = 16
B, D, H, W = 2, 160, 192, 224
SPLIT = 4
HH = H // SPLIT
NUM_TILES = B * D * SPLIT
TILES_PER_WORKER = NUM_TILES // NUM_WORKERS
VECS_PER_ROW = W // LANES
NBUF = 5
TBL = 32


@functools.partial(
    pl.kernel,
    out_type=jax.ShapeDtypeStruct((B, 1, D, H, W), jnp.float32),
    mesh=plsc.VectorSubcoreMesh(
        core_axis_name="c",
        subcore_axis_name="s",
        num_cores=NUM_CORES,
        num_subcores=NUM_SUBCORES,
    ),
    scratch_types=[
        pltpu.VMEM((TBL, LANES), jnp.float32),
        pltpu.VMEM((TBL,), jnp.int32),
        pltpu.VMEM((TBL,), jnp.int32),
        [pltpu.VMEM((HH, W), jnp.float32) for _ in range(NBUF)],
        [pltpu.VMEM((HH, W), jnp.float32) for _ in range(NBUF)],
        [pltpu.SemaphoreType.DMA for _ in range(NBUF)],
        [pltpu.SemaphoreType.DMA for _ in range(NBUF)],
    ],
    compiler_params=pltpu.CompilerParams(needs_layout_passes=False),
)
def _remap(x_hbm, src_hbm, dst_hbm, out_hbm, tab_v, src_v, dst_v, inb, outb,
           in_sem, out_sem):
    wid = lax.axis_index("s") * NUM_CORES + lax.axis_index("c")
    tile0 = wid * TILES_PER_WORKER

    pltpu.sync_copy(src_hbm, src_v)
    pltpu.sync_copy(dst_hbm, dst_v)
    lane = lax.iota(jnp.int32, LANES)
    zero = jnp.zeros((LANES,), jnp.float32)
    for v in range(TBL):
        tab_v[v, pl.ds(0, LANES)] = zero
    for h in range(TBL // LANES):
        s = src_v[pl.ds(h * LANES, LANES)]
        d = dst_v[pl.ds(h * LANES, LANES)].astype(jnp.float32)
        for l in range(LANES):
            plsc.store_scatter(tab_v, [s, jnp.full((LANES,), l, jnp.int32)],
                               d)

    def tile_slice(ref, t):
        n = t // (D * SPLIT)
        r = t % (D * SPLIT)
        d = r // SPLIT
        h = (r % SPLIT) * HH
        return ref.at[n, 0, d, pl.ds(h, HH)]

    for b in range(NBUF):
        pltpu.async_copy(tile_slice(x_hbm, tile0 + b), inb[b], in_sem[b])

    @pl.loop(0, TILES_PER_WORKER, step=NBUF)
    def _(c0):
        for b in range(NBUF):
            c = c0 + b
            t = tile0 + c
            pltpu.make_async_copy(tile_slice(x_hbm, t), inb[b],
                                  in_sem[b]).wait()

            @pl.when(c >= NBUF)
            def _():
                pltpu.make_async_copy(outb[b], tile_slice(out_hbm, t),
                                      out_sem[b]).wait()

            src_buf = inb[b]
            dst_buf = outb[b]

            @plsc.parallel_loop(0, HH, 1, unroll=2)
            def _(r):
                for j in range(VECS_PER_ROW):
                    v = src_buf[r, pl.ds(j * LANES, LANES)]
                    idx = v.astype(jnp.int32)
                    dst_buf[r, pl.ds(j * LANES, LANES)] = plsc.load_gather(
                        tab_v, [idx, lane])

            pltpu.async_copy(dst_buf, tile_slice(out_hbm, t), out_sem[b])

            @pl.when(c + NBUF < TILES_PER_WORKER)
            def _():
                pltpu.async_copy(tile_slice(x_hbm, t + NBUF), inb[b],
                                 in_sem[b])

    for b in range(NBUF):
        t = tile0 + TILES_PER_WORKER - NBUF + b
        pltpu.make_async_copy(outb[b], tile_slice(out_hbm, t),
                              out_sem[b]).wait()


def kernel(x, source_values, dest_values):
    return _remap(x, source_values, dest_values)


# speedup vs baseline: 1.2372x; 1.0364x over previous
import functools

import jax
import jax.numpy as jnp
from jax import lax
from jax.experimental import pallas as pl
from jax.experimental.pallas import tpu as pltpu
from jax.experimental.pallas import tpu_sc as plsc

NUM_CORES = 2
NUM_SUBCORES = 16
NUM_WORKERS = NUM_CORES * NUM_SUBCORES
LANES = 16
B, D, H, W = 2, 160, 192, 224
SPLIT = 4
HH = H // SPLIT
NUM_TILES = B * D * SPLIT
TILES_PER_WORKER = NUM_TILES // NUM_WORKERS
VECS_PER_ROW = W // LANES
NBUF = 4
TBL = 32


@functools.partial(
    pl.kernel,
    out_type=jax.ShapeDtypeStruct((B, 1, D, H, W), jnp.float32),
    mesh=plsc.VectorSubcoreMesh(
        core_axis_name="c",
        subcore_axis_name="s",
        num_cores=NUM_CORES,
        num_subcores=NUM_SUBCORES,
    ),
    scratch_types=[
        pltpu.VMEM((TBL, LANES), jnp.float32),
        pltpu.VMEM((TBL,), jnp.int32),
        pltpu.VMEM((TBL,), jnp.int32),
        [pltpu.VMEM((HH, W), jnp.float32) for _ in range(NBUF)],
        [pltpu.VMEM((HH, W), jnp.float32) for _ in range(NBUF)],
        [pltpu.SemaphoreType.DMA for _ in range(NBUF)],
        [pltpu.SemaphoreType.DMA for _ in range(NBUF)],
    ],
    compiler_params=pltpu.CompilerParams(needs_layout_passes=False),
)
def _remap(x_hbm, src_hbm, dst_hbm, out_hbm, tab_v, src_v, dst_v, inb, outb,
           in_sem, out_sem):
    wid = lax.axis_index("s") * NUM_CORES + lax.axis_index("c")
    tile0 = wid * TILES_PER_WORKER

    pltpu.sync_copy(src_hbm, src_v)
    pltpu.sync_copy(dst_hbm, dst_v)
    lane = lax.iota(jnp.int32, LANES)
    zero = jnp.zeros((LANES,), jnp.float32)
    for v in range(TBL):
        tab_v[v, pl.ds(0, LANES)] = zero
    for h in range(TBL // LANES):
        s = src_v[pl.ds(h * LANES, LANES)]
        d = dst_v[pl.ds(h * LANES, LANES)].astype(jnp.float32)
        for l in range(LANES):
            plsc.store_scatter(tab_v, [s, jnp.full((LANES,), l, jnp.int32)],
                               d)

    def tile_slice(ref, t):
        n = t // (D * SPLIT)
        r = t % (D * SPLIT)
        d = r // SPLIT
        h = (r % SPLIT) * HH
        return ref.at[n, 0, d, pl.ds(h, HH)]

    for b in range(NBUF):
        pltpu.async_copy(tile_slice(x_hbm, tile0 + b), inb[b], in_sem[b])

    @pl.loop(0, TILES_PER_WORKER, step=NBUF)
    def _(c0):
        for b in range(NBUF):
            c = c0 + b
            t = tile0 + c
            pltpu.make_async_copy(tile_slice(x_hbm, t), inb[b],
                                  in_sem[b]).wait()

            @pl.when(c >= NBUF)
            def _():
                pltpu.make_async_copy(outb[b], tile_slice(out_hbm, t),
                                      out_sem[b]).wait()

            src_buf = inb[b]
            dst_buf = outb[b]

            @plsc.parallel_loop(0, HH, 1, unroll=3)
            def _(r):
                for j in range(VECS_PER_ROW):
                    v = src_buf[r, pl.ds(j * LANES, LANES)]
                    idx = v.astype(jnp.int32)
                    dst_buf[r, pl.ds(j * LANES, LANES)] = plsc.load_gather(
                        tab_v, [idx, lane])

            pltpu.async_copy(dst_buf, tile_slice(out_hbm, t), out_sem[b])

            @pl.when(c + NBUF < TILES_PER_WORKER)
            def _():
                pltpu.async_copy(tile_slice(x_hbm, t + NBUF), inb[b],
                                 in_sem[b])

    for b in range(NBUF):
        t = tile0 + TILES_PER_WORKER - NBUF + b
        pltpu.make_async_copy(outb[b], tile_slice(out_hbm, t),
                              out_sem[b]).wait()


def kernel(x, source_values, dest_values):
    return _remap(x, source_values, dest_values)
